# baseline (speedup 1.0000x reference)
import functools

import jax
import jax.numpy as jnp
from jax import lax
from jax.experimental import pallas as pl
from jax.experimental.pallas import tpu as pltpu
from jax.experimental.pallas import tpu_sc as plsc

F32 = jnp.float32
NC = 2
NS = 16
CHUNK = 128
HG = 64


def _mesh():
    return plsc.VectorSubcoreMesh(
        core_axis_name="c", subcore_axis_name="s", num_cores=NC, num_subcores=NS
    )


def _make_deg_kernel(nrows, nchunks):
    rpt = nrows // NS

    def body(dst_hbm, out_hbm, idx_v, obuf, acc):
        c = lax.axis_index("c")
        s = lax.axis_index("s")

        def fill(i, _):
            obuf[i, :] = jnp.ones((16,), F32)
            return 0

        lax.fori_loop(0, CHUNK, fill, 0)
        pltpu.sync_copy(dst_hbm.at[c, s], idx_v)
        base = s * rpt
        done = 0
        while done < rpt:
            step = min(CHUNK, rpt - done)
            pltpu.sync_copy(obuf.at[pl.ds(0, step)], acc.at[pl.ds(base + done, step)])
            done += step
        plsc.subcore_barrier()

        def scat(j, _):
            pltpu.sync_copy(obuf, acc.at[idx_v.at[j]], add=True)
            return 0

        lax.fori_loop(0, nchunks, scat, 0)
        plsc.subcore_barrier()
        pltpu.sync_copy(acc.at[pl.ds(base, rpt)], out_hbm.at[c, pl.ds(base, rpt)])

    return pl.kernel(
        body,
        out_type=jax.ShapeDtypeStruct((NC, nrows, 16), F32),
        mesh=_mesh(),
        scratch_types=[
            pltpu.VMEM((nchunks, CHUNK), jnp.int32),
            pltpu.VMEM((CHUNK, 16), F32),
            pltpu.VMEM_SHARED((nrows, 16), F32),
        ],
    )


def _make_scatter_kernel(nrows, nchunks, grp, tbase_mult):
    rpt = nrows // NS
    ngroups = nchunks // grp
    npairs = grp // 2

    def body(table, src_hbm, dst_hbm, out_hbm, srcv, dstv, rb0, rb1, acc,
             sg0, sg1, ss0, ss1):
        c = lax.axis_index("c")
        s = lax.axis_index("s")
        base = s * rpt
        tb = c * tbase_mult + base
        pltpu.sync_copy(table.at[pl.ds(tb, rpt)], acc.at[pl.ds(base, rpt)])
        plsc.subcore_barrier()

        def gather(j, rb, sg):
            for o in range(0, CHUNK, HG):
                pltpu.async_copy(
                    table.at[srcv.at[j, pl.ds(o, HG)]], rb.at[pl.ds(o, HG)],
                    sg)

        def wait_gather(j, rb, sg):
            for o in range(0, CHUNK, HG):
                pltpu.make_async_copy(
                    table.at[srcv.at[j, pl.ds(o, HG)]], rb.at[pl.ds(o, HG)],
                    sg).wait()

        def group(g, _):
            gb = pl.multiple_of(g * grp, grp)
            pltpu.sync_copy(src_hbm.at[c, s, pl.ds(gb, grp)], srcv)
            pltpu.sync_copy(dst_hbm.at[c, s, pl.ds(gb, grp)], dstv)
            gather(0, rb0, sg0)
            gather(1, rb1, sg1)

            def pair(p, _):
                j0 = 2 * p
                wait_gather(j0, rb0, sg0)
                pltpu.async_copy(rb0, acc.at[dstv.at[j0]], ss0, add=True)
                wait_gather(j0 + 1, rb1, sg1)
                pltpu.async_copy(rb1, acc.at[dstv.at[j0 + 1]], ss1, add=True)
                pltpu.make_async_copy(rb0, acc.at[dstv.at[j0]], ss0).wait()

                @pl.when(j0 + 2 < grp)
                def _():
                    gather(j0 + 2, rb0, sg0)

                pltpu.make_async_copy(
                    rb1, acc.at[dstv.at[j0 + 1]], ss1).wait()

                @pl.when(j0 + 3 < grp)
                def _():
                    gather(j0 + 3, rb1, sg1)

                return 0

            lax.fori_loop(0, npairs, pair, 0)
            return 0

        lax.fori_loop(0, ngroups, group, 0)
        plsc.subcore_barrier()
        pltpu.sync_copy(acc.at[pl.ds(base, rpt)], out_hbm.at[c, pl.ds(base, rpt)])

    return pl.kernel(
        body,
        out_type=jax.ShapeDtypeStruct((NC, nrows, 128), F32),
        mesh=_mesh(),
        scratch_types=[
            pltpu.VMEM((grp, CHUNK), jnp.int32),
            pltpu.VMEM((grp, CHUNK), jnp.int32),
            pltpu.VMEM((CHUNK, 128), F32),
            pltpu.VMEM((CHUNK, 128), F32),
            pltpu.VMEM_SHARED((nrows, 128), F32),
            pltpu.SemaphoreType.DMA,
            pltpu.SemaphoreType.DMA,
            pltpu.SemaphoreType.DMA,
            pltpu.SemaphoreType.DMA,
        ],
    )


def _dis_of(dg):
    d = dg[...]
    return lax.rsqrt(d[0, :, 0:1] + d[1, :, 0:1] - 1.0)


def _stack_halves(xv, out_ref):
    w = out_ref.shape[-1]
    out_ref[...] = jnp.stack([xv[:, i * w:(i + 1) * w]
                              for i in range(xv.shape[1] // w)])


def _pre_body(degT, xpad, xp_out):
    xp_out[...] = xpad[...] * _dis_of(degT)


def _acc_stats(h, sum_out, sq_out):
    @pl.when(pl.program_id(0) == 0)
    def _():
        sum_out[...] = jnp.zeros_like(sum_out)
        sq_out[...] = jnp.zeros_like(sq_out)

    sum_out[...] += jnp.sum(h, axis=0, keepdims=True)
    sq_out[...] += jnp.sum(h * h, axis=0, keepdims=True)


def _a1_body(P, degT, xb, w, b, wres, bres, h_out, id1_out, sum_out, sq_out):
    dis = _dis_of(degT)
    x = xb[...]
    agg = dis * (P[0] + P[1] - dis * x)
    h = jnp.dot(agg, w[...], preferred_element_type=F32) + b[...]
    h_out[...] = h
    id1_out[...] = jnp.dot(x, wres[...], preferred_element_type=F32) + bres[...]
    _acc_stats(h, sum_out, sq_out)


def _a23_body(P, degT, w, b, h_out, sum_out, sq_out):
    Pv = P[...]
    agg = _dis_of(degT) * jnp.concatenate([Pv[0], Pv[1]], axis=1)
    h = jnp.dot(agg, w[...], preferred_element_type=F32) + b[...]
    h_out[...] = h
    _acc_stats(h, sum_out, sq_out)


def _b_body(h_ref, res_ref, degT, sum_ref, sq_ref, g, be, x_out, hv_out, *,
            n, emit_halves):
    h = h_ref[...]
    mu = sum_ref[...] * (1.0 / n)
    var = sq_ref[...] * (1.0 / n) - mu * mu
    xn = (
        jax.nn.relu(g[...] * (h - mu) * lax.rsqrt(var + 1e-5) + be[...])
        + res_ref[...]
    )
    x_out[...] = xn
    if emit_halves:
        _stack_halves(xn * _dis_of(degT), hv_out)


def kernel(x, edge_index, W1, b1, g1, be1, Wres, bres, W2, b2, g2, be2,
           W3, b3, g3, be3):
    n, din = x.shape
    dh = W1.shape[1]
    e = edge_index.shape[1]
    nrows = -(-(n + 1) // 128) * 128
    gr = n

    src = edge_index[0]
    dst = edge_index[1]

    def lay(idx, per_tile_groups, ntiles, padval):
        tot = per_tile_groups * CHUNK * ntiles
        p = jnp.pad(idx, (0, tot - e), constant_values=padval)
        return p.reshape(ntiles, per_tile_groups, CHUNK)

    ch32 = -(-(-(-e // (NC * NS * CHUNK))) // 40) * 40
    ch16 = -(-(-(-e // (NS * CHUNK))) // 40) * 40
    src32 = lay(src, ch32, NC * NS, 0).reshape(NC, NS, ch32, CHUNK)
    dst32 = lay(dst, ch32, NC * NS, gr).reshape(NC, NS, ch32, CHUNK)
    s16 = lay(src, ch16, NS, 0)
    src16 = jnp.stack([s16, s16 + nrows])
    d16 = lay(dst, ch16, NS, gr)
    dst16 = jnp.stack([d16, d16])

    xpad = jnp.pad(x, ((0, nrows - n), (0, 0)))

    deg_call = _make_deg_kernel(nrows, ch32)
    scat1 = _make_scatter_kernel(nrows, ch32, 40, 0)
    scat23 = _make_scatter_kernel(nrows, ch16, 40, nrows)

    degT = deg_call(dst32)

    R = 2000
    RP = nrows // 8
    gsteps = n // R

    def rows(bs, im=None):
        return pl.BlockSpec(bs, im if im is not None else (lambda i: (i, 0)))

    wspec = lambda shp: pl.BlockSpec(shp, lambda i: (0, 0))
    vec = lambda d: pl.BlockSpec((d,), lambda i: (0,))
    stat = pl.BlockSpec((1, dh), lambda i: (0, 0))
    p_spec = pl.BlockSpec((NC, R, 128), lambda i: (0, i, 0))
    degT_spec = pl.BlockSpec((NC, R, 16), lambda i: (0, i, 0))

    pre = pl.pallas_call(
        _pre_body,
        grid=(nrows // RP,),
        in_specs=[pl.BlockSpec((NC, RP, 16), lambda i: (0, i, 0)), rows((RP, din))],
        out_specs=rows((RP, din)),
        out_shape=jax.ShapeDtypeStruct((nrows, din), F32),
    )
    xp = pre(degT, xpad)

    P1 = scat1(xp, src32, dst32)

    a1 = pl.pallas_call(
        _a1_body,
        grid=(gsteps,),
        in_specs=[p_spec, degT_spec, rows((R, din)),
                  wspec((din, dh)), vec(dh), wspec((din, dh)), vec(dh)],
        out_specs=(rows((R, dh)), rows((R, dh)), stat, stat),
        out_shape=(
            jax.ShapeDtypeStruct((n, dh), F32),
            jax.ShapeDtypeStruct((n, dh), F32),
            jax.ShapeDtypeStruct((1, dh), F32),
            jax.ShapeDtypeStruct((1, dh), F32),
        ),
    )

    a23 = pl.pallas_call(
        _a23_body,
        grid=(gsteps,),
        in_specs=[p_spec, degT_spec, wspec((dh, dh)), vec(dh)],
        out_specs=(rows((R, dh)), stat, stat),
        out_shape=(
            jax.ShapeDtypeStruct((n, dh), F32),
            jax.ShapeDtypeStruct((1, dh), F32),
            jax.ShapeDtypeStruct((1, dh), F32),
        ),
    )

    hv_spec = pl.BlockSpec((NC, R, 128), lambda i: (0, i, 0))

    def make_b(emit_halves):
        outs = [jax.ShapeDtypeStruct((n, dh), F32)]
        ospecs = [rows((R, dh))]
        if emit_halves:
            outs.append(jax.ShapeDtypeStruct((NC, nrows, 128), F32))
            ospecs.append(hv_spec)
        if emit_halves:
            body = functools.partial(_b_body, n=n, emit_halves=True)
        else:
            def body(h, r, dg, sm, sq, g, be, xo):
                _b_body(h, r, dg, sm, sq, g, be, xo, None,
                        n=n, emit_halves=False)
        return pl.pallas_call(
            body,
            grid=(gsteps,),
            in_specs=[rows((R, dh)), rows((R, dh)), degT_spec, stat, stat,
                      vec(dh), vec(dh)],
            out_specs=tuple(ospecs) if emit_halves else ospecs[0],
            out_shape=tuple(outs) if emit_halves else outs[0],
        )

    b_mid = make_b(True)
    b_last = make_b(False)

    h1, id1, s1, q1 = a1(P1, degT, x, W1, b1, Wres, bres)
    x1, h1v = b_mid(h1, id1, degT, s1, q1, g1, be1)

    P2 = scat23(h1v.reshape(NC * nrows, 128), src16, dst16)
    h2, s2, q2 = a23(P2, degT, W2, b2)
    x2, h2v = b_mid(h2, x1, degT, s2, q2, g2, be2)

    P3 = scat23(h2v.reshape(NC * nrows, 128), src16, dst16)
    h3, s3, q3 = a23(P3, degT, W3, b3)
    x3 = b_last(h3, x2, degT, s3, q3, g3, be3)
    return (x1, x2, x3)


# speedup vs baseline: 1.0560x; 1.0560x over previous
import functools

import jax
import jax.numpy as jnp
from jax import lax
from jax.experimental import pallas as pl
from jax.experimental.pallas import tpu as pltpu
from jax.experimental.pallas import tpu_sc as plsc

F32 = jnp.float32
NC = 2
NS = 16
CHUNK = 128
HG = 64


def _mesh():
    return plsc.VectorSubcoreMesh(
        core_axis_name="c", subcore_axis_name="s", num_cores=NC, num_subcores=NS
    )


def _make_deg_kernel(nrows, nchunks):
    rpt = nrows // NS

    def body(dst_hbm, out_hbm, idx_v, obuf, acc):
        c = lax.axis_index("c")
        s = lax.axis_index("s")

        def fill(i, _):
            obuf[i, :] = jnp.ones((16,), F32)
            return 0

        lax.fori_loop(0, CHUNK, fill, 0)
        pltpu.sync_copy(dst_hbm.at[c, s], idx_v)
        base = s * rpt
        done = 0
        while done < rpt:
            step = min(CHUNK, rpt - done)
            pltpu.sync_copy(obuf.at[pl.ds(0, step)], acc.at[pl.ds(base + done, step)])
            done += step
        plsc.subcore_barrier()

        def scat(j, _):
            pltpu.sync_copy(obuf, acc.at[idx_v.at[j]], add=True)
            return 0

        lax.fori_loop(0, nchunks, scat, 0)
        plsc.subcore_barrier()
        pltpu.sync_copy(acc.at[pl.ds(base, rpt)], out_hbm.at[c, pl.ds(base, rpt)])

    return pl.kernel(
        body,
        out_type=jax.ShapeDtypeStruct((NC, nrows, 16), F32),
        mesh=_mesh(),
        scratch_types=[
            pltpu.VMEM((nchunks, CHUNK), jnp.int32),
            pltpu.VMEM((CHUNK, 16), F32),
            pltpu.VMEM_SHARED((nrows, 16), F32),
        ],
    )


def _make_scatter_kernel(nrows, nchunks, grp, tbase_mult):
    rpt = nrows // NS
    ngroups = nchunks // grp
    npairs = grp // 2

    def body(table, src_hbm, dst_hbm, out_hbm, srcv, dstv, rb0, rb1, acc,
             sg0, sg1):
        c = lax.axis_index("c")
        s = lax.axis_index("s")
        base = s * rpt
        tb = c * tbase_mult + base
        pltpu.sync_copy(table.at[pl.ds(tb, rpt)], acc.at[pl.ds(base, rpt)])
        plsc.subcore_barrier()

        def gather(j, rb, sg):
            for o in range(0, CHUNK, HG):
                pltpu.async_copy(
                    table.at[srcv.at[j, pl.ds(o, HG)]], rb.at[pl.ds(o, HG)],
                    sg)

        def wait_gather(j, rb, sg):
            for o in range(0, CHUNK, HG):
                pltpu.make_async_copy(
                    table.at[srcv.at[j, pl.ds(o, HG)]], rb.at[pl.ds(o, HG)],
                    sg).wait()

        def group(g, _):
            gb = pl.multiple_of(g * grp, grp)
            pltpu.sync_copy(src_hbm.at[c, s, pl.ds(gb, grp)], srcv)
            pltpu.sync_copy(dst_hbm.at[c, s, pl.ds(gb, grp)], dstv)
            gather(0, rb0, sg0)
            gather(1, rb1, sg1)

            def pair(p, _):
                j0 = 2 * p
                wait_gather(j0, rb0, sg0)

                @pl.when(j0 + 2 < grp)
                def _():
                    gather(j0 + 2, rb0, sg0)

                pltpu.sync_copy(rb0, acc.at[dstv.at[j0]], add=True)
                wait_gather(j0 + 1, rb1, sg1)

                @pl.when(j0 + 3 < grp)
                def _():
                    gather(j0 + 3, rb1, sg1)

                pltpu.sync_copy(rb1, acc.at[dstv.at[j0 + 1]], add=True)
                return 0

            lax.fori_loop(0, npairs, pair, 0)
            return 0

        lax.fori_loop(0, ngroups, group, 0)
        plsc.subcore_barrier()
        pltpu.sync_copy(acc.at[pl.ds(base, rpt)], out_hbm.at[c, pl.ds(base, rpt)])

    return pl.kernel(
        body,
        out_type=jax.ShapeDtypeStruct((NC, nrows, 128), F32),
        mesh=_mesh(),
        scratch_types=[
            pltpu.VMEM((grp, CHUNK), jnp.int32),
            pltpu.VMEM((grp, CHUNK), jnp.int32),
            pltpu.VMEM((CHUNK, 128), F32),
            pltpu.VMEM((CHUNK, 128), F32),
            pltpu.VMEM_SHARED((nrows, 128), F32),
            pltpu.SemaphoreType.DMA,
            pltpu.SemaphoreType.DMA,
        ],
    )


def _dis_of(dg):
    d = dg[...]
    return lax.rsqrt(d[0, :, 0:1] + d[1, :, 0:1] - 1.0)


def _stack_halves(xv, out_ref):
    w = out_ref.shape[-1]
    out_ref[...] = jnp.stack([xv[:, i * w:(i + 1) * w]
                              for i in range(xv.shape[1] // w)])


def _pre_body(degT, xpad, xp_out):
    xp_out[...] = xpad[...] * _dis_of(degT)


def _acc_stats(h, sum_out, sq_out):
    @pl.when(pl.program_id(0) == 0)
    def _():
        sum_out[...] = jnp.zeros_like(sum_out)
        sq_out[...] = jnp.zeros_like(sq_out)

    sum_out[...] += jnp.sum(h, axis=0, keepdims=True)
    sq_out[...] += jnp.sum(h * h, axis=0, keepdims=True)


def _a1_body(P, degT, xb, w, b, wres, bres, h_out, id1_out, sum_out, sq_out):
    dis = _dis_of(degT)
    x = xb[...]
    agg = dis * (P[0] + P[1] - dis * x)
    h = jnp.dot(agg, w[...], preferred_element_type=F32) + b[...]
    h_out[...] = h
    id1_out[...] = jnp.dot(x, wres[...], preferred_element_type=F32) + bres[...]
    _acc_stats(h, sum_out, sq_out)


def _a23_body(P, degT, w, b, h_out, sum_out, sq_out):
    Pv = P[...]
    agg = _dis_of(degT) * jnp.concatenate([Pv[0], Pv[1]], axis=1)
    h = jnp.dot(agg, w[...], preferred_element_type=F32) + b[...]
    h_out[...] = h
    _acc_stats(h, sum_out, sq_out)


def _b_body(h_ref, res_ref, degT, sum_ref, sq_ref, g, be, x_out, hv_out, *,
            n, emit_halves):
    h = h_ref[...]
    mu = sum_ref[...] * (1.0 / n)
    var = sq_ref[...] * (1.0 / n) - mu * mu
    xn = (
        jax.nn.relu(g[...] * (h - mu) * lax.rsqrt(var + 1e-5) + be[...])
        + res_ref[...]
    )
    x_out[...] = xn
    if emit_halves:
        _stack_halves(xn * _dis_of(degT), hv_out)


def kernel(x, edge_index, W1, b1, g1, be1, Wres, bres, W2, b2, g2, be2,
           W3, b3, g3, be3):
    n, din = x.shape
    dh = W1.shape[1]
    e = edge_index.shape[1]
    nrows = -(-(n + 1) // 128) * 128
    gr = n

    src = edge_index[0]
    dst = edge_index[1]

    def lay(idx, per_tile_groups, ntiles, padval):
        tot = per_tile_groups * CHUNK * ntiles
        p = jnp.pad(idx, (0, tot - e), constant_values=padval)
        return p.reshape(ntiles, per_tile_groups, CHUNK)

    ch32 = -(-(-(-e // (NC * NS * CHUNK))) // 40) * 40
    ch16 = -(-(-(-e // (NS * CHUNK))) // 40) * 40
    src32 = lay(src, ch32, NC * NS, 0).reshape(NC, NS, ch32, CHUNK)
    dst32 = lay(dst, ch32, NC * NS, gr).reshape(NC, NS, ch32, CHUNK)
    s16 = lay(src, ch16, NS, 0)
    src16 = jnp.stack([s16, s16 + nrows])
    d16 = lay(dst, ch16, NS, gr)
    dst16 = jnp.stack([d16, d16])

    xpad = jnp.pad(x, ((0, nrows - n), (0, 0)))

    deg_call = _make_deg_kernel(nrows, ch32)
    scat1 = _make_scatter_kernel(nrows, ch32, 40, 0)
    scat23 = _make_scatter_kernel(nrows, ch16, 40, nrows)

    degT = deg_call(dst32)

    R = 2000
    RP = nrows // 8
    gsteps = n // R

    def rows(bs, im=None):
        return pl.BlockSpec(bs, im if im is not None else (lambda i: (i, 0)))

    wspec = lambda shp: pl.BlockSpec(shp, lambda i: (0, 0))
    vec = lambda d: pl.BlockSpec((d,), lambda i: (0,))
    stat = pl.BlockSpec((1, dh), lambda i: (0, 0))
    p_spec = pl.BlockSpec((NC, R, 128), lambda i: (0, i, 0))
    degT_spec = pl.BlockSpec((NC, R, 16), lambda i: (0, i, 0))

    pre = pl.pallas_call(
        _pre_body,
        grid=(nrows // RP,),
        in_specs=[pl.BlockSpec((NC, RP, 16), lambda i: (0, i, 0)), rows((RP, din))],
        out_specs=rows((RP, din)),
        out_shape=jax.ShapeDtypeStruct((nrows, din), F32),
    )
    xp = pre(degT, xpad)

    P1 = scat1(xp, src32, dst32)

    a1 = pl.pallas_call(
        _a1_body,
        grid=(gsteps,),
        in_specs=[p_spec, degT_spec, rows((R, din)),
                  wspec((din, dh)), vec(dh), wspec((din, dh)), vec(dh)],
        out_specs=(rows((R, dh)), rows((R, dh)), stat, stat),
        out_shape=(
            jax.ShapeDtypeStruct((n, dh), F32),
            jax.ShapeDtypeStruct((n, dh), F32),
            jax.ShapeDtypeStruct((1, dh), F32),
            jax.ShapeDtypeStruct((1, dh), F32),
        ),
    )

    a23 = pl.pallas_call(
        _a23_body,
        grid=(gsteps,),
        in_specs=[p_spec, degT_spec, wspec((dh, dh)), vec(dh)],
        out_specs=(rows((R, dh)), stat, stat),
        out_shape=(
            jax.ShapeDtypeStruct((n, dh), F32),
            jax.ShapeDtypeStruct((1, dh), F32),
            jax.ShapeDtypeStruct((1, dh), F32),
        ),
    )

    hv_spec = pl.BlockSpec((NC, R, 128), lambda i: (0, i, 0))

    def make_b(emit_halves):
        outs = [jax.ShapeDtypeStruct((n, dh), F32)]
        ospecs = [rows((R, dh))]
        if emit_halves:
            outs.append(jax.ShapeDtypeStruct((NC, nrows, 128), F32))
            ospecs.append(hv_spec)
        if emit_halves:
            body = functools.partial(_b_body, n=n, emit_halves=True)
        else:
            def body(h, r, dg, sm, sq, g, be, xo):
                _b_body(h, r, dg, sm, sq, g, be, xo, None,
                        n=n, emit_halves=False)
        return pl.pallas_call(
            body,
            grid=(gsteps,),
            in_specs=[rows((R, dh)), rows((R, dh)), degT_spec, stat, stat,
                      vec(dh), vec(dh)],
            out_specs=tuple(ospecs) if emit_halves else ospecs[0],
            out_shape=tuple(outs) if emit_halves else outs[0],
        )

    b_mid = make_b(True)
    b_last = make_b(False)

    h1, id1, s1, q1 = a1(P1, degT, x, W1, b1, Wres, bres)
    x1, h1v = b_mid(h1, id1, degT, s1, q1, g1, be1)

    P2 = scat23(h1v.reshape(NC * nrows, 128), src16, dst16)
    h2, s2, q2 = a23(P2, degT, W2, b2)
    x2, h2v = b_mid(h2, x1, degT, s2, q2, g2, be2)

    P3 = scat23(h2v.reshape(NC * nrows, 128), src16, dst16)
    h3, s3, q3 = a23(P3, degT, W3, b3)
    x3 = b_last(h3, x2, degT, s3, q3, g3, be3)
    return (x1, x2, x3)
